# baseline (speedup 1.0000x reference)
import functools

import jax
import jax.numpy as jnp
from jax import lax
from jax.experimental import pallas as pl
from jax.experimental.pallas import tpu as pltpu
from jax.experimental.pallas import tpu_sc as plsc

E = 64
D = 768
H = 128
C = 128
B = 2048
LN_EPS = 1e-5
RB = 128
NB = B // RB
T = NB + E
RBR = 256
NC, NS = 2, 16
NW = NC * NS
NTOK = B // NW

_F32 = jnp.float32
_PREC = lax.Precision.DEFAULT
_SC_PARAMS = pltpu.CompilerParams(needs_layout_passes=False)


def _gelu(v):
    return 0.5 * v * (1.0 + lax.erf(v * 0.7071067811865476))


def _iota16():
    return lax.broadcasted_iota(jnp.int32, (16,), 0)


def _router_body(x_ref, Wr_ref, br_ref, eid_ref, xhat_ref):
    logits = lax.dot_general(x_ref[...], Wr_ref[...], (((1,), (1,)), ((), ())),
                             precision=_PREC, preferred_element_type=_F32)
    logits = logits + br_ref[...]
    eid_ref[...] = jnp.argmax(logits, axis=1).astype(jnp.int32)[:, None]
    xv = x_ref[...]
    mu = jnp.mean(xv, axis=1, keepdims=True)
    var = jnp.mean((xv - mu) ** 2, axis=1, keepdims=True)
    xhat_ref[...] = (xv - mu) * lax.rsqrt(var + LN_EPS)


def _router(x, Wr, br):
    return pl.pallas_call(
        _router_body,
        grid=(B // RBR,),
        in_specs=[
            pl.BlockSpec((RBR, D), lambda i: (i, 0)),
            pl.BlockSpec((E, D), lambda i: (0, 0)),
            pl.BlockSpec((1, E), lambda i: (0, 0)),
        ],
        out_specs=[pl.BlockSpec((RBR, 1), lambda i: (i, 0)),
                   pl.BlockSpec((RBR, D), lambda i: (i, 0))],
        out_shape=[jax.ShapeDtypeStruct((B, 1), jnp.int32),
                   jax.ShapeDtypeStruct((B, D), _F32)],
    )(x, Wr, br.reshape(1, E))


def _sc_hist_body(eid_hbm, hists_hbm, ranks_hbm, eidv, hist, rank):
    w = lax.axis_index("s") * NC + lax.axis_index("c")
    base = w * NTOK
    pltpu.sync_copy(eid_hbm.at[pl.ds(base, NTOK)], eidv)
    z16 = jnp.zeros((16,), jnp.int32)
    for k in range(E // 16):
        hist[pl.ds(16 * k, 16)] = z16

    lane0 = _iota16() == 0

    def body(i, carry):
        e16 = plsc.load_gather(eidv, [jnp.full((16,), i, jnp.int32)])
        r16 = plsc.load_gather(hist, [e16])
        plsc.store_scatter(rank, [jnp.full((16,), i, jnp.int32)], r16,
                           mask=lane0)
        plsc.store_scatter(hist, [e16], r16 + 1, mask=lane0)
        return carry

    lax.fori_loop(0, NTOK, body, 0)
    pltpu.sync_copy(hist, hists_hbm.at[w])
    pltpu.sync_copy(rank, ranks_hbm.at[pl.ds(base, NTOK)])


def _sc_scatter_body(xhat_hbm, eid_hbm, hists_hbm, ranks_hbm,
                     xs_hbm, invperm_hbm, rbs_hbm, es_hbm, los_hbm, his_hbm,
                     firsts_hbm,
                     allc, eidv, rankv, destv, basev, cnts,
                     lob_v, tbase_v, es_arr, meta_v, xrows, sem):
    w = lax.axis_index("s") * NC + lax.axis_index("c")
    base = w * NTOK
    xfetch = pltpu.async_copy(xhat_hbm.at[pl.ds(base, NTOK)], xrows, sem)
    pltpu.sync_copy(hists_hbm, allc)
    pltpu.sync_copy(eid_hbm.at[pl.ds(base, NTOK)], eidv)
    pltpu.sync_copy(ranks_hbm.at[pl.ds(base, NTOK)], rankv)

    carry = jnp.int32(0)
    for k in range(E // 16):
        z16 = jnp.zeros((16,), jnp.int32)

        def wbody(w2, tm):
            tot, mine = tm
            row = allc[w2, pl.ds(16 * k, 16)]
            tot = tot + row
            mine = mine + jnp.where(w2 < w, row, 0)
            return (tot, mine)

        tot, mine = lax.fori_loop(0, NW, wbody, (z16, z16))
        cs = plsc.cumsum(tot)
        basev[pl.ds(16 * k, 16)] = cs - tot + carry + mine
        cnts[pl.ds(16 * k, 16)] = tot
        carry = carry + jnp.sum(tot)

    for k in range(NTOK // 16):
        ev = eidv[pl.ds(16 * k, 16)]
        bg = plsc.load_gather(basev, [ev])
        destv[pl.ds(16 * k, 16)] = bg + rankv[pl.ds(16 * k, 16)]

    pltpu.sync_copy(destv, invperm_hbm.at[pl.ds(base, NTOK)])

    @pl.when(w == 0)
    def _pair_schedule():
        z16 = jnp.zeros((16,), jnp.int32)
        for k in range(T // 16):
            es_arr[pl.ds(16 * k, 16)] = z16
        tcar = jnp.int32(0)
        for k in range(E // 16):
            off = basev[pl.ds(16 * k, 16)]
            cnt = cnts[pl.ds(16 * k, 16)]
            lob = lax.div(off, RB)
            hib = lax.div(off + cnt - 1, RB)
            nb = jnp.where(cnt > 0, hib - lob + 1, 0)
            cs = plsc.cumsum(nb)
            tb = cs - nb + tcar
            tcar = tcar + jnp.sum(nb)
            lob_v[pl.ds(16 * k, 16)] = lob
            tbase_v[pl.ds(16 * k, 16)] = tb
            plsc.store_scatter(es_arr, [tb], _iota16() + 16 * k,
                               mask=cnt > 0)
        mcar = jnp.int32(0)
        for k in range(T // 16):
            ev = es_arr[pl.ds(16 * k, 16)]
            es = jnp.maximum(plsc.cummax(ev), mcar)
            mcar = jnp.max(es)
            tv = _iota16() + 16 * k
            lobk = plsc.load_gather(lob_v, [es])
            tbk = plsc.load_gather(tbase_v, [es])
            offk = plsc.load_gather(basev, [es])
            cntk = plsc.load_gather(cnts, [es])
            rb = lobk + (tv - tbk)
            lo = jnp.maximum(offk, rb * RB)
            hi = jnp.minimum(offk + cntk, rb * RB + RB)
            validv = tv < tcar
            rb = jnp.where(validv, rb, NB - 1)
            lo = jnp.where(validv, lo, 0)
            hi = jnp.where(validv, hi, 0)
            fr = jnp.where(lo == rb * RB, 1, 0)
            meta_v[0, pl.ds(16 * k, 16)] = rb
            meta_v[1, pl.ds(16 * k, 16)] = jnp.where(validv, es, 0)
            meta_v[2, pl.ds(16 * k, 16)] = lo
            meta_v[3, pl.ds(16 * k, 16)] = hi
            meta_v[4, pl.ds(16 * k, 16)] = fr
        pltpu.sync_copy(meta_v.at[0], rbs_hbm)
        pltpu.sync_copy(meta_v.at[1], es_hbm)
        pltpu.sync_copy(meta_v.at[2], los_hbm)
        pltpu.sync_copy(meta_v.at[3], his_hbm)
        pltpu.sync_copy(meta_v.at[4], firsts_hbm)

    xfetch.wait()
    pltpu.async_copy(xrows, xs_hbm.at[destv], sem).wait()


def _sc_dispatch(xhat, eid):
    mesh = plsc.VectorSubcoreMesh(core_axis_name="c", subcore_axis_name="s")
    hists, ranks = pl.kernel(
        _sc_hist_body,
        out_type=[jax.ShapeDtypeStruct((NW, E), jnp.int32),
                  jax.ShapeDtypeStruct((B,), jnp.int32)],
        mesh=mesh,
        compiler_params=_SC_PARAMS,
        scratch_types=[pltpu.VMEM((NTOK,), jnp.int32),
                       pltpu.VMEM((E,), jnp.int32),
                       pltpu.VMEM((NTOK,), jnp.int32)],
    )(eid)
    outs = pl.kernel(
        _sc_scatter_body,
        out_type=[jax.ShapeDtypeStruct((B, D), _F32),
                  jax.ShapeDtypeStruct((B,), jnp.int32)] +
                 [jax.ShapeDtypeStruct((T,), jnp.int32)] * 5,
        mesh=mesh,
        compiler_params=_SC_PARAMS,
        scratch_types=[pltpu.VMEM((NW, E), jnp.int32),
                       pltpu.VMEM((NTOK,), jnp.int32),
                       pltpu.VMEM((NTOK,), jnp.int32),
                       pltpu.VMEM((NTOK,), jnp.int32),
                       pltpu.VMEM((E,), jnp.int32),
                       pltpu.VMEM((E,), jnp.int32),
                       pltpu.VMEM((E,), jnp.int32),
                       pltpu.VMEM((E,), jnp.int32),
                       pltpu.VMEM((T,), jnp.int32),
                       pltpu.VMEM((5, T), jnp.int32),
                       pltpu.VMEM((NTOK, D), _F32),
                       pltpu.SemaphoreType.DMA],
    )(xhat, eid, hists, ranks)
    xs, invperm = outs[0], outs[1]
    meta = tuple(outs[2:])
    return xs, invperm, meta


def _gmm_body(rbs_r, es_r, los_r, his_r, first_r,
              xs_ref, gamma_ref, beta_ref, W1_ref, b1_ref, W2_ref, b2_ref,
              out_ref):
    t = pl.program_id(0)
    lo = los_r[t]
    hi = his_r[t]

    @pl.when(first_r[t] != 0)
    def _init():
        out_ref[...] = jnp.zeros_like(out_ref)

    @pl.when(hi > lo)
    def _compute():
        xn = xs_ref[...] * gamma_ref[0] + beta_ref[0]
        h = _gelu(lax.dot_general(xn, W1_ref[0], (((1,), (0,)), ((), ())),
                                  precision=_PREC, preferred_element_type=_F32)
                  + b1_ref[0])
        y = lax.dot_general(h, W2_ref[0], (((1,), (0,)), ((), ())),
                            precision=_PREC, preferred_element_type=_F32) + b2_ref[0]
        row = rbs_r[t] * RB + lax.broadcasted_iota(jnp.int32, (RB, 1), 0)
        y = jnp.where((row >= lo) & (row < hi), y, 0.0)
        out_ref[...] = out_ref[...] + y


def _gmm(meta, xs, gamma, beta, W1, b1, W2, b2):
    grid_spec = pltpu.PrefetchScalarGridSpec(
        num_scalar_prefetch=5,
        grid=(T,),
        in_specs=[
            pl.BlockSpec((RB, D), lambda t, rbs, es, los, his, fs: (rbs[t], 0)),
            pl.BlockSpec((1, 1, D), lambda t, rbs, es, los, his, fs: (es[t], 0, 0)),
            pl.BlockSpec((1, 1, D), lambda t, rbs, es, los, his, fs: (es[t], 0, 0)),
            pl.BlockSpec((1, D, H), lambda t, rbs, es, los, his, fs: (es[t], 0, 0)),
            pl.BlockSpec((1, 1, H), lambda t, rbs, es, los, his, fs: (es[t], 0, 0)),
            pl.BlockSpec((1, H, C), lambda t, rbs, es, los, his, fs: (es[t], 0, 0)),
            pl.BlockSpec((1, 1, C), lambda t, rbs, es, los, his, fs: (es[t], 0, 0)),
        ],
        out_specs=pl.BlockSpec((RB, C), lambda t, rbs, es, los, his, fs: (rbs[t], 0)),
    )
    return pl.pallas_call(
        _gmm_body,
        grid_spec=grid_spec,
        out_shape=jax.ShapeDtypeStruct((B, C), _F32),
    )(*meta, xs, gamma.reshape(E, 1, D), beta.reshape(E, 1, D),
      W1, b1.reshape(E, 1, H), W2, b2.reshape(E, 1, C))


def _sc_unperm_body(ys_hbm, inv_hbm, out_hbm, ipv, yrows, sem):
    w = lax.axis_index("s") * NC + lax.axis_index("c")
    base = w * NTOK
    pltpu.sync_copy(inv_hbm.at[pl.ds(base, NTOK)], ipv)
    pltpu.async_copy(ys_hbm.at[ipv], yrows, sem).wait()
    pltpu.sync_copy(yrows, out_hbm.at[pl.ds(base, NTOK)])


def _sc_unpermute(ys, invperm):
    mesh = plsc.VectorSubcoreMesh(core_axis_name="c", subcore_axis_name="s")
    return pl.kernel(
        _sc_unperm_body,
        out_type=jax.ShapeDtypeStruct((B, C), _F32),
        mesh=mesh,
        compiler_params=_SC_PARAMS,
        scratch_types=[pltpu.VMEM((NTOK,), jnp.int32),
                       pltpu.VMEM((NTOK, C), _F32),
                       pltpu.SemaphoreType.DMA],
    )(ys, invperm)


def kernel(x, Wr, br, gamma, beta, W1, b1, W2, b2):
    eid, xhat = _router(x, Wr, br)
    xs, invperm, meta = _sc_dispatch(xhat, eid.reshape(B))
    ys = _gmm(meta, xs, gamma, beta, W1, b1, W2, b2)
    return _sc_unpermute(ys, invperm)


# speedup vs baseline: 1.0147x; 1.0147x over previous
import functools

import jax
import jax.numpy as jnp
from jax import lax
from jax.experimental import pallas as pl
from jax.experimental.pallas import tpu as pltpu
from jax.experimental.pallas import tpu_sc as plsc

E = 64
D = 768
H = 128
C = 128
B = 2048
LN_EPS = 1e-5
RB = 128
NB = B // RB
T = NB + E
RBR = 256
NC, NS = 2, 16
NW = NC * NS
NTOK = B // NW

_F32 = jnp.float32
_PREC = lax.Precision.DEFAULT
_SC_PARAMS = pltpu.CompilerParams(needs_layout_passes=False)


def _gelu(v):
    return 0.5 * v * (1.0 + lax.erf(v * 0.7071067811865476))


def _iota16():
    return lax.broadcasted_iota(jnp.int32, (16,), 0)


def _router_body(x_ref, Wr_ref, br_ref, eid_ref, xhat_ref):
    logits = lax.dot_general(x_ref[...], Wr_ref[...], (((1,), (1,)), ((), ())),
                             precision=_PREC, preferred_element_type=_F32)
    logits = logits + br_ref[...]
    eid_ref[...] = jnp.argmax(logits, axis=1).astype(jnp.int32)
    xv = x_ref[...]
    mu = jnp.mean(xv, axis=1, keepdims=True)
    var = jnp.mean((xv - mu) ** 2, axis=1, keepdims=True)
    xhat_ref[...] = (xv - mu) * lax.rsqrt(var + LN_EPS)


def _router(x, Wr, br):
    return pl.pallas_call(
        _router_body,
        grid=(B // RBR,),
        in_specs=[
            pl.BlockSpec((RBR, D), lambda i: (i, 0)),
            pl.BlockSpec((E, D), lambda i: (0, 0)),
            pl.BlockSpec((1, E), lambda i: (0, 0)),
        ],
        out_specs=[pl.BlockSpec((RBR,), lambda i: (i,)),
                   pl.BlockSpec((RBR, D), lambda i: (i, 0))],
        out_shape=[jax.ShapeDtypeStruct((B,), jnp.int32),
                   jax.ShapeDtypeStruct((B, D), _F32)],
    )(x, Wr, br.reshape(1, E))


def _sc_hist_body(eid_hbm, hists_hbm, ranks_hbm, eidv, hist, rank):
    w = lax.axis_index("s") * NC + lax.axis_index("c")
    base = w * NTOK
    pltpu.sync_copy(eid_hbm.at[pl.ds(base, NTOK)], eidv)
    z16 = jnp.zeros((16,), jnp.int32)
    for k in range(E // 16):
        hist[pl.ds(16 * k, 16)] = z16

    lane0 = _iota16() == 0

    def body(i, carry):
        e16 = plsc.load_gather(eidv, [jnp.full((16,), i, jnp.int32)])
        r16 = plsc.load_gather(hist, [e16])
        plsc.store_scatter(rank, [jnp.full((16,), i, jnp.int32)], r16,
                           mask=lane0)
        plsc.store_scatter(hist, [e16], r16 + 1, mask=lane0)
        return carry

    lax.fori_loop(0, NTOK, body, 0)
    pltpu.sync_copy(hist, hists_hbm.at[w])
    pltpu.sync_copy(rank, ranks_hbm.at[pl.ds(base, NTOK)])


def _sc_scatter_body(xhat_hbm, eid_hbm, hists_hbm, ranks_hbm,
                     xs_hbm, invperm_hbm, rbs_hbm, es_hbm, los_hbm, his_hbm,
                     firsts_hbm,
                     allc, eidv, rankv, destv, basev, cnts,
                     lob_v, tbase_v, es_arr, meta_v, xrows, sem):
    w = lax.axis_index("s") * NC + lax.axis_index("c")
    base = w * NTOK
    xfetch = pltpu.async_copy(xhat_hbm.at[pl.ds(base, NTOK)], xrows, sem)
    pltpu.sync_copy(hists_hbm, allc)
    pltpu.sync_copy(eid_hbm.at[pl.ds(base, NTOK)], eidv)
    pltpu.sync_copy(ranks_hbm.at[pl.ds(base, NTOK)], rankv)

    carry = jnp.int32(0)
    for k in range(E // 16):
        z16 = jnp.zeros((16,), jnp.int32)

        def wbody(w2, tm):
            tot, mine = tm
            row = allc[w2, pl.ds(16 * k, 16)]
            tot = tot + row
            mine = mine + jnp.where(w2 < w, row, 0)
            return (tot, mine)

        tot, mine = lax.fori_loop(0, NW, wbody, (z16, z16))
        cs = plsc.cumsum(tot)
        basev[pl.ds(16 * k, 16)] = cs - tot + carry + mine
        cnts[pl.ds(16 * k, 16)] = tot
        carry = carry + jnp.sum(tot)

    for k in range(NTOK // 16):
        ev = eidv[pl.ds(16 * k, 16)]
        bg = plsc.load_gather(basev, [ev])
        destv[pl.ds(16 * k, 16)] = bg + rankv[pl.ds(16 * k, 16)]

    pltpu.sync_copy(destv, invperm_hbm.at[pl.ds(base, NTOK)])

    @pl.when(w == 0)
    def _pair_schedule():
        z16 = jnp.zeros((16,), jnp.int32)
        for k in range(T // 16):
            es_arr[pl.ds(16 * k, 16)] = z16
        tcar = jnp.int32(0)
        for k in range(E // 16):
            off = basev[pl.ds(16 * k, 16)]
            cnt = cnts[pl.ds(16 * k, 16)]
            lob = lax.div(off, RB)
            hib = lax.div(off + cnt - 1, RB)
            nb = jnp.where(cnt > 0, hib - lob + 1, 0)
            cs = plsc.cumsum(nb)
            tb = cs - nb + tcar
            tcar = tcar + jnp.sum(nb)
            lob_v[pl.ds(16 * k, 16)] = lob
            tbase_v[pl.ds(16 * k, 16)] = tb
            plsc.store_scatter(es_arr, [tb], _iota16() + 16 * k,
                               mask=cnt > 0)
        mcar = jnp.int32(0)
        for k in range(T // 16):
            ev = es_arr[pl.ds(16 * k, 16)]
            es = jnp.maximum(plsc.cummax(ev), mcar)
            mcar = jnp.max(es)
            tv = _iota16() + 16 * k
            lobk = plsc.load_gather(lob_v, [es])
            tbk = plsc.load_gather(tbase_v, [es])
            offk = plsc.load_gather(basev, [es])
            cntk = plsc.load_gather(cnts, [es])
            rb = lobk + (tv - tbk)
            lo = jnp.maximum(offk, rb * RB)
            hi = jnp.minimum(offk + cntk, rb * RB + RB)
            validv = tv < tcar
            rb = jnp.where(validv, rb, NB - 1)
            lo = jnp.where(validv, lo, 0)
            hi = jnp.where(validv, hi, 0)
            fr = jnp.where(lo == rb * RB, 1, 0)
            meta_v[0, pl.ds(16 * k, 16)] = rb
            meta_v[1, pl.ds(16 * k, 16)] = jnp.where(validv, es, 0)
            meta_v[2, pl.ds(16 * k, 16)] = lo
            meta_v[3, pl.ds(16 * k, 16)] = hi
            meta_v[4, pl.ds(16 * k, 16)] = fr
        pltpu.sync_copy(meta_v.at[0], rbs_hbm)
        pltpu.sync_copy(meta_v.at[1], es_hbm)
        pltpu.sync_copy(meta_v.at[2], los_hbm)
        pltpu.sync_copy(meta_v.at[3], his_hbm)
        pltpu.sync_copy(meta_v.at[4], firsts_hbm)

    xfetch.wait()
    pltpu.async_copy(xrows, xs_hbm.at[destv], sem).wait()


def _sc_dispatch(xhat, eid):
    mesh = plsc.VectorSubcoreMesh(core_axis_name="c", subcore_axis_name="s")
    hists, ranks = pl.kernel(
        _sc_hist_body,
        out_type=[jax.ShapeDtypeStruct((NW, E), jnp.int32),
                  jax.ShapeDtypeStruct((B,), jnp.int32)],
        mesh=mesh,
        compiler_params=_SC_PARAMS,
        scratch_types=[pltpu.VMEM((NTOK,), jnp.int32),
                       pltpu.VMEM((E,), jnp.int32),
                       pltpu.VMEM((NTOK,), jnp.int32)],
    )(eid)
    outs = pl.kernel(
        _sc_scatter_body,
        out_type=[jax.ShapeDtypeStruct((B, D), _F32),
                  jax.ShapeDtypeStruct((B,), jnp.int32)] +
                 [jax.ShapeDtypeStruct((T,), jnp.int32)] * 5,
        mesh=mesh,
        compiler_params=_SC_PARAMS,
        scratch_types=[pltpu.VMEM((NW, E), jnp.int32),
                       pltpu.VMEM((NTOK,), jnp.int32),
                       pltpu.VMEM((NTOK,), jnp.int32),
                       pltpu.VMEM((NTOK,), jnp.int32),
                       pltpu.VMEM((E,), jnp.int32),
                       pltpu.VMEM((E,), jnp.int32),
                       pltpu.VMEM((E,), jnp.int32),
                       pltpu.VMEM((E,), jnp.int32),
                       pltpu.VMEM((T,), jnp.int32),
                       pltpu.VMEM((5, T), jnp.int32),
                       pltpu.VMEM((NTOK, D), _F32),
                       pltpu.SemaphoreType.DMA],
    )(xhat, eid, hists, ranks)
    xs, invperm = outs[0], outs[1]
    meta = tuple(outs[2:])
    return xs, invperm, meta


def _gmm_body(rbs_r, es_r, los_r, his_r, first_r,
              xs_ref, gamma_ref, beta_ref, W1_ref, b1_ref, W2_ref, b2_ref,
              out_ref):
    t = pl.program_id(0)
    lo = los_r[t]
    hi = his_r[t]

    @pl.when(hi > lo)
    def _compute():
        xn = xs_ref[...] * gamma_ref[0] + beta_ref[0]
        h = _gelu(lax.dot_general(xn, W1_ref[0], (((1,), (0,)), ((), ())),
                                  precision=_PREC, preferred_element_type=_F32)
                  + b1_ref[0])
        y = lax.dot_general(h, W2_ref[0], (((1,), (0,)), ((), ())),
                            precision=_PREC, preferred_element_type=_F32) + b2_ref[0]
        row = rbs_r[t] * RB + lax.broadcasted_iota(jnp.int32, (RB, 1), 0)
        y = jnp.where((row >= lo) & (row < hi), y, 0.0)
        out_ref[...] = jnp.where(first_r[t] != 0, y, out_ref[...] + y)


def _gmm(meta, xs, gamma, beta, W1, b1, W2, b2):
    grid_spec = pltpu.PrefetchScalarGridSpec(
        num_scalar_prefetch=5,
        grid=(T,),
        in_specs=[
            pl.BlockSpec((RB, D), lambda t, rbs, es, los, his, fs: (rbs[t], 0)),
            pl.BlockSpec((1, 1, D), lambda t, rbs, es, los, his, fs: (es[t], 0, 0)),
            pl.BlockSpec((1, 1, D), lambda t, rbs, es, los, his, fs: (es[t], 0, 0)),
            pl.BlockSpec((1, D, H), lambda t, rbs, es, los, his, fs: (es[t], 0, 0)),
            pl.BlockSpec((1, 1, H), lambda t, rbs, es, los, his, fs: (es[t], 0, 0)),
            pl.BlockSpec((1, H, C), lambda t, rbs, es, los, his, fs: (es[t], 0, 0)),
            pl.BlockSpec((1, 1, C), lambda t, rbs, es, los, his, fs: (es[t], 0, 0)),
        ],
        out_specs=pl.BlockSpec((RB, C), lambda t, rbs, es, los, his, fs: (rbs[t], 0)),
    )
    return pl.pallas_call(
        _gmm_body,
        grid_spec=grid_spec,
        out_shape=jax.ShapeDtypeStruct((B, C), _F32),
    )(*meta, xs, gamma.reshape(E, 1, D), beta.reshape(E, 1, D),
      W1, b1.reshape(E, 1, H), W2, b2.reshape(E, 1, C))


def _sc_unperm_body(ys_hbm, inv_hbm, out_hbm, ipv, yrows, sem):
    w = lax.axis_index("s") * NC + lax.axis_index("c")
    base = w * NTOK
    pltpu.sync_copy(inv_hbm.at[pl.ds(base, NTOK)], ipv)
    pltpu.async_copy(ys_hbm.at[ipv], yrows, sem).wait()
    pltpu.sync_copy(yrows, out_hbm.at[pl.ds(base, NTOK)])


def _sc_unpermute(ys, invperm):
    mesh = plsc.VectorSubcoreMesh(core_axis_name="c", subcore_axis_name="s")
    return pl.kernel(
        _sc_unperm_body,
        out_type=jax.ShapeDtypeStruct((B, C), _F32),
        mesh=mesh,
        compiler_params=_SC_PARAMS,
        scratch_types=[pltpu.VMEM((NTOK,), jnp.int32),
                       pltpu.VMEM((NTOK, C), _F32),
                       pltpu.SemaphoreType.DMA],
    )(ys, invperm)


def kernel(x, Wr, br, gamma, beta, W1, b1, W2, b2):
    eid, xhat = _router(x, Wr, br)
    xs, invperm, meta = _sc_dispatch(xhat, eid)
    ys = _gmm(meta, xs, gamma, beta, W1, b1, W2, b2)
    return _sc_unpermute(ys, invperm)


# speedup vs baseline: 1.0328x; 1.0178x over previous
import functools

import jax
import jax.numpy as jnp
from jax import lax
from jax.experimental import pallas as pl
from jax.experimental.pallas import tpu as pltpu
from jax.experimental.pallas import tpu_sc as plsc

E = 64
D = 768
H = 128
C = 128
B = 2048
LN_EPS = 1e-5
RB = 128
NB = B // RB
T = NB + E
RBR = 256
NC, NS = 2, 16
NW = NC * NS
NTOK = B // NW

_F32 = jnp.float32
_PREC = lax.Precision.DEFAULT
_SC_PARAMS = pltpu.CompilerParams(needs_layout_passes=False)


def _gelu(v):
    return 0.5 * v * (1.0 + lax.erf(v * 0.7071067811865476))


def _iota16():
    return lax.broadcasted_iota(jnp.int32, (16,), 0)


def _router_body(x_ref, Wr_ref, br_ref, eid_ref, xhat_ref):
    logits = lax.dot_general(x_ref[...], Wr_ref[...], (((1,), (1,)), ((), ())),
                             precision=_PREC, preferred_element_type=_F32)
    logits = logits + br_ref[...]
    eid_ref[...] = jnp.argmax(logits, axis=1).astype(jnp.int32)
    xv = x_ref[...]
    mu = jnp.mean(xv, axis=1, keepdims=True)
    var = jnp.mean((xv - mu) ** 2, axis=1, keepdims=True)
    xhat_ref[...] = (xv - mu) * lax.rsqrt(var + LN_EPS)


def _router(x, Wr, br):
    return pl.pallas_call(
        _router_body,
        grid=(B // RBR,),
        in_specs=[
            pl.BlockSpec((RBR, D), lambda i: (i, 0)),
            pl.BlockSpec((E, D), lambda i: (0, 0)),
            pl.BlockSpec((1, E), lambda i: (0, 0)),
        ],
        out_specs=[pl.BlockSpec((RBR,), lambda i: (i,)),
                   pl.BlockSpec((RBR, D), lambda i: (i, 0))],
        out_shape=[jax.ShapeDtypeStruct((B,), jnp.int32),
                   jax.ShapeDtypeStruct((B, D), _F32)],
    )(x, Wr, br.reshape(1, E))


def _sc_hist_body(eid_hbm, hists_hbm, ranks_hbm, eidv, hist, rank):
    w = lax.axis_index("s") * NC + lax.axis_index("c")
    base = w * NTOK
    pltpu.sync_copy(eid_hbm.at[pl.ds(base, NTOK)], eidv)
    z16 = jnp.zeros((16,), jnp.int32)
    for k in range(E // 16):
        hist[pl.ds(16 * k, 16)] = z16

    lane0 = _iota16() == 0

    def body(i, carry):
        e16 = plsc.load_gather(eidv, [jnp.full((16,), i, jnp.int32)])
        r16 = plsc.load_gather(hist, [e16])
        plsc.store_scatter(rank, [jnp.full((16,), i, jnp.int32)], r16,
                           mask=lane0)
        plsc.store_scatter(hist, [e16], r16 + 1, mask=lane0)
        return carry

    lax.fori_loop(0, NTOK, body, 0)
    pltpu.sync_copy(hist, hists_hbm.at[w])
    pltpu.sync_copy(rank, ranks_hbm.at[pl.ds(base, NTOK)])


def _sc_scatter_body(xhat_hbm, eid_hbm, hists_hbm, ranks_hbm,
                     xs_hbm, invperm_hbm, meta_hbm,
                     allc, eidv, rankv, destv, basev, cnts,
                     lob_v, tbase_v, es_arr, meta_v, xrows, sem):
    w = lax.axis_index("s") * NC + lax.axis_index("c")
    base = w * NTOK
    xfetch = pltpu.async_copy(xhat_hbm.at[pl.ds(base, NTOK)], xrows, sem)
    pltpu.sync_copy(hists_hbm, allc)
    pltpu.sync_copy(eid_hbm.at[pl.ds(base, NTOK)], eidv)
    pltpu.sync_copy(ranks_hbm.at[pl.ds(base, NTOK)], rankv)

    carry = jnp.int32(0)
    for k in range(E // 16):
        z16 = jnp.zeros((16,), jnp.int32)

        def wbody(w2, tm):
            tot, mine = tm
            row = allc[w2, pl.ds(16 * k, 16)]
            tot = tot + row
            mine = mine + jnp.where(w2 < w, row, 0)
            return (tot, mine)

        tot, mine = lax.fori_loop(0, NW, wbody, (z16, z16))
        cs = plsc.cumsum(tot)
        basev[pl.ds(16 * k, 16)] = cs - tot + carry + mine
        cnts[pl.ds(16 * k, 16)] = tot
        carry = carry + jnp.sum(tot)

    for k in range(NTOK // 16):
        ev = eidv[pl.ds(16 * k, 16)]
        bg = plsc.load_gather(basev, [ev])
        destv[pl.ds(16 * k, 16)] = bg + rankv[pl.ds(16 * k, 16)]

    pltpu.sync_copy(destv, invperm_hbm.at[pl.ds(base, NTOK)])

    @pl.when(w == 0)
    def _pair_schedule():
        z16 = jnp.zeros((16,), jnp.int32)
        for k in range(T // 16):
            es_arr[pl.ds(16 * k, 16)] = z16
        tcar = jnp.int32(0)
        for k in range(E // 16):
            off = basev[pl.ds(16 * k, 16)]
            cnt = cnts[pl.ds(16 * k, 16)]
            lob = lax.div(off, RB)
            hib = lax.div(off + cnt - 1, RB)
            nb = jnp.where(cnt > 0, hib - lob + 1, 0)
            cs = plsc.cumsum(nb)
            tb = cs - nb + tcar
            tcar = tcar + jnp.sum(nb)
            lob_v[pl.ds(16 * k, 16)] = lob
            tbase_v[pl.ds(16 * k, 16)] = tb
            plsc.store_scatter(es_arr, [tb], _iota16() + 16 * k,
                               mask=cnt > 0)
        mcar = jnp.int32(0)
        for k in range(T // 16):
            ev = es_arr[pl.ds(16 * k, 16)]
            es = jnp.maximum(plsc.cummax(ev), mcar)
            mcar = jnp.max(es)
            tv = _iota16() + 16 * k
            lobk = plsc.load_gather(lob_v, [es])
            tbk = plsc.load_gather(tbase_v, [es])
            offk = plsc.load_gather(basev, [es])
            cntk = plsc.load_gather(cnts, [es])
            rb = lobk + (tv - tbk)
            lo = jnp.maximum(offk, rb * RB)
            hi = jnp.minimum(offk + cntk, rb * RB + RB)
            validv = tv < tcar
            rb = jnp.where(validv, rb, NB - 1)
            lo = jnp.where(validv, lo, 0)
            hi = jnp.where(validv, hi, 0)
            fr = jnp.where(lo == rb * RB, 1, 0)
            meta_v[0, pl.ds(16 * k, 16)] = rb
            meta_v[1, pl.ds(16 * k, 16)] = jnp.where(validv, es, 0)
            meta_v[2, pl.ds(16 * k, 16)] = lo
            meta_v[3, pl.ds(16 * k, 16)] = hi
            meta_v[4, pl.ds(16 * k, 16)] = fr
        pltpu.sync_copy(meta_v, meta_hbm)

    xfetch.wait()
    pltpu.async_copy(xrows, xs_hbm.at[destv], sem).wait()


def _sc_dispatch(xhat, eid):
    mesh = plsc.VectorSubcoreMesh(core_axis_name="c", subcore_axis_name="s")
    hists, ranks = pl.kernel(
        _sc_hist_body,
        out_type=[jax.ShapeDtypeStruct((NW, E), jnp.int32),
                  jax.ShapeDtypeStruct((B,), jnp.int32)],
        mesh=mesh,
        compiler_params=_SC_PARAMS,
        scratch_types=[pltpu.VMEM((NTOK,), jnp.int32),
                       pltpu.VMEM((E,), jnp.int32),
                       pltpu.VMEM((NTOK,), jnp.int32)],
    )(eid)
    outs = pl.kernel(
        _sc_scatter_body,
        out_type=[jax.ShapeDtypeStruct((B, D), _F32),
                  jax.ShapeDtypeStruct((B,), jnp.int32),
                  jax.ShapeDtypeStruct((5, T), jnp.int32)],
        mesh=mesh,
        compiler_params=_SC_PARAMS,
        scratch_types=[pltpu.VMEM((NW, E), jnp.int32),
                       pltpu.VMEM((NTOK,), jnp.int32),
                       pltpu.VMEM((NTOK,), jnp.int32),
                       pltpu.VMEM((NTOK,), jnp.int32),
                       pltpu.VMEM((E,), jnp.int32),
                       pltpu.VMEM((E,), jnp.int32),
                       pltpu.VMEM((E,), jnp.int32),
                       pltpu.VMEM((E,), jnp.int32),
                       pltpu.VMEM((T,), jnp.int32),
                       pltpu.VMEM((5, T), jnp.int32),
                       pltpu.VMEM((NTOK, D), _F32),
                       pltpu.SemaphoreType.DMA],
    )(xhat, eid, hists, ranks)
    return outs[0], outs[1], outs[2]


def _gmm_body(m_ref,
              xs_ref, gamma_ref, beta_ref, W1_ref, b1_ref, W2_ref, b2_ref,
              out_ref):
    t = pl.program_id(0)
    lo = m_ref[2, t]
    hi = m_ref[3, t]

    @pl.when(hi > lo)
    def _compute():
        base = m_ref[0, t] * RB
        xn = xs_ref[pl.ds(base, RB), :] * gamma_ref[0] + beta_ref[0]
        h = _gelu(lax.dot_general(xn, W1_ref[0], (((1,), (0,)), ((), ())),
                                  precision=_PREC, preferred_element_type=_F32)
                  + b1_ref[0])
        y = lax.dot_general(h, W2_ref[0], (((1,), (0,)), ((), ())),
                            precision=_PREC, preferred_element_type=_F32) + b2_ref[0]
        row = base + lax.broadcasted_iota(jnp.int32, (RB, 1), 0)
        y = jnp.where((row >= lo) & (row < hi), y, 0.0)
        prev = jnp.where(m_ref[4, t] != 0, 0.0, out_ref[pl.ds(base, RB), :])
        out_ref[pl.ds(base, RB), :] = prev + y


def _gmm(meta, xs, gamma, beta, W1, b1, W2, b2):
    grid_spec = pltpu.PrefetchScalarGridSpec(
        num_scalar_prefetch=1,
        grid=(T,),
        in_specs=[
            pl.BlockSpec((B, D), lambda t, m: (0, 0)),
            pl.BlockSpec((1, 1, D), lambda t, m: (m[1, t], 0, 0)),
            pl.BlockSpec((1, 1, D), lambda t, m: (m[1, t], 0, 0)),
            pl.BlockSpec((1, D, H), lambda t, m: (m[1, t], 0, 0)),
            pl.BlockSpec((1, 1, H), lambda t, m: (m[1, t], 0, 0)),
            pl.BlockSpec((1, H, C), lambda t, m: (m[1, t], 0, 0)),
            pl.BlockSpec((1, 1, C), lambda t, m: (m[1, t], 0, 0)),
        ],
        out_specs=pl.BlockSpec((B, C), lambda t, m: (0, 0)),
    )
    return pl.pallas_call(
        _gmm_body,
        grid_spec=grid_spec,
        out_shape=jax.ShapeDtypeStruct((B, C), _F32),
    )(meta, xs, gamma.reshape(E, 1, D), beta.reshape(E, 1, D),
      W1, b1.reshape(E, 1, H), W2, b2.reshape(E, 1, C))


def _sc_unperm_body(ys_hbm, inv_hbm, out_hbm, ipv, yrows, sem):
    w = lax.axis_index("s") * NC + lax.axis_index("c")
    base = w * NTOK
    pltpu.sync_copy(inv_hbm.at[pl.ds(base, NTOK)], ipv)
    pltpu.async_copy(ys_hbm.at[ipv], yrows, sem).wait()
    pltpu.sync_copy(yrows, out_hbm.at[pl.ds(base, NTOK)])


def _sc_unpermute(ys, invperm):
    mesh = plsc.VectorSubcoreMesh(core_axis_name="c", subcore_axis_name="s")
    return pl.kernel(
        _sc_unperm_body,
        out_type=jax.ShapeDtypeStruct((B, C), _F32),
        mesh=mesh,
        compiler_params=_SC_PARAMS,
        scratch_types=[pltpu.VMEM((NTOK,), jnp.int32),
                       pltpu.VMEM((NTOK, C), _F32),
                       pltpu.SemaphoreType.DMA],
    )(ys, invperm)


def kernel(x, Wr, br, gamma, beta, W1, b1, W2, b2):
    eid, xhat = _router(x, Wr, br)
    xs, invperm, meta = _sc_dispatch(xhat, eid)
    ys = _gmm(meta, xs, gamma, beta, W1, b1, W2, b2)
    return _sc_unpermute(ys, invperm)


# speedup vs baseline: 1.0383x; 1.0054x over previous
import functools

import jax
import jax.numpy as jnp
from jax import lax
from jax.experimental import pallas as pl
from jax.experimental.pallas import tpu as pltpu
from jax.experimental.pallas import tpu_sc as plsc

E = 64
D = 768
H = 128
C = 128
B = 2048
LN_EPS = 1e-5
RB = 128
NB = B // RB
T = NB + E
RBR = 256
NC, NS = 2, 16
NW = NC * NS
NTOK = B // NW

_F32 = jnp.float32
_PREC = lax.Precision.DEFAULT
_SC_PARAMS = pltpu.CompilerParams(needs_layout_passes=False)


def _gelu(v):
    return 0.5 * v * (1.0 + lax.erf(v * 0.7071067811865476))


def _iota16():
    return lax.broadcasted_iota(jnp.int32, (16,), 0)


def _router_body(x_ref, Wr_ref, br_ref, eid_ref, xhat_ref):
    logits = lax.dot_general(x_ref[...], Wr_ref[...], (((1,), (1,)), ((), ())),
                             precision=_PREC, preferred_element_type=_F32)
    logits = logits + br_ref[...]
    eid_ref[...] = jnp.argmax(logits, axis=1).astype(jnp.int32)
    xv = x_ref[...]
    mu = jnp.mean(xv, axis=1, keepdims=True)
    var = jnp.mean((xv - mu) ** 2, axis=1, keepdims=True)
    xhat_ref[...] = (xv - mu) * lax.rsqrt(var + LN_EPS)


def _router(x, Wr, br):
    return pl.pallas_call(
        _router_body,
        grid=(B // RBR,),
        in_specs=[
            pl.BlockSpec((RBR, D), lambda i: (i, 0)),
            pl.BlockSpec((E, D), lambda i: (0, 0)),
            pl.BlockSpec((1, E), lambda i: (0, 0)),
        ],
        out_specs=[pl.BlockSpec((RBR,), lambda i: (i,)),
                   pl.BlockSpec((RBR, D), lambda i: (i, 0))],
        out_shape=[jax.ShapeDtypeStruct((B,), jnp.int32),
                   jax.ShapeDtypeStruct((B, D), _F32)],
    )(x, Wr, br.reshape(1, E))


def _sc_hist_body(eid_hbm, hists_hbm, ranks_hbm, eidv, hist, rank):
    w = lax.axis_index("s") * NC + lax.axis_index("c")
    base = w * NTOK
    pltpu.sync_copy(eid_hbm.at[pl.ds(base, NTOK)], eidv)
    z16 = jnp.zeros((16,), jnp.int32)
    for k in range(E // 16):
        hist[pl.ds(16 * k, 16)] = z16

    lane0 = _iota16() == 0

    def body(i, carry):
        e16 = plsc.load_gather(eidv, [jnp.full((16,), i, jnp.int32)])
        r16 = plsc.load_gather(hist, [e16])
        plsc.store_scatter(rank, [jnp.full((16,), i, jnp.int32)], r16,
                           mask=lane0)
        plsc.store_scatter(hist, [e16], r16 + 1, mask=lane0)
        return carry

    lax.fori_loop(0, NTOK, body, 0)
    pltpu.sync_copy(hist, hists_hbm.at[w])
    pltpu.sync_copy(rank, ranks_hbm.at[pl.ds(base, NTOK)])


def _sc_scatter_body(xhat_hbm, eid_hbm, hists_hbm, ranks_hbm,
                     xs_hbm, invperm_hbm, meta_hbm,
                     allc, eidv, rankv, destv, basev, cnts,
                     lob_v, tbase_v, es_arr, meta_v, xrows, sem):
    w = lax.axis_index("s") * NC + lax.axis_index("c")
    base = w * NTOK
    xfetch = pltpu.async_copy(xhat_hbm.at[pl.ds(base, NTOK)], xrows, sem)
    pltpu.sync_copy(hists_hbm, allc)
    pltpu.sync_copy(eid_hbm.at[pl.ds(base, NTOK)], eidv)
    pltpu.sync_copy(ranks_hbm.at[pl.ds(base, NTOK)], rankv)

    carry = jnp.int32(0)
    for k in range(E // 16):
        z16 = jnp.zeros((16,), jnp.int32)

        def wbody(w2, tm):
            tot, mine = tm
            row = allc[w2, pl.ds(16 * k, 16)]
            tot = tot + row
            mine = mine + jnp.where(w2 < w, row, 0)
            return (tot, mine)

        tot, mine = lax.fori_loop(0, NW, wbody, (z16, z16))
        cs = plsc.cumsum(tot)
        basev[pl.ds(16 * k, 16)] = cs - tot + carry + mine
        cnts[pl.ds(16 * k, 16)] = tot
        carry = carry + jnp.sum(tot)

    for k in range(NTOK // 16):
        ev = eidv[pl.ds(16 * k, 16)]
        bg = plsc.load_gather(basev, [ev])
        destv[pl.ds(16 * k, 16)] = bg + rankv[pl.ds(16 * k, 16)]

    pltpu.sync_copy(destv, invperm_hbm.at[pl.ds(base, NTOK)])

    @pl.when(w == 0)
    def _pair_schedule():
        z16 = jnp.zeros((16,), jnp.int32)
        for k in range(T // 16):
            es_arr[pl.ds(16 * k, 16)] = z16
        tcar = jnp.int32(0)
        for k in range(E // 16):
            off = basev[pl.ds(16 * k, 16)]
            cnt = cnts[pl.ds(16 * k, 16)]
            lob = lax.div(off, RB)
            hib = lax.div(off + cnt - 1, RB)
            nb = jnp.where(cnt > 0, hib - lob + 1, 0)
            cs = plsc.cumsum(nb)
            tb = cs - nb + tcar
            tcar = tcar + jnp.sum(nb)
            lob_v[pl.ds(16 * k, 16)] = lob
            tbase_v[pl.ds(16 * k, 16)] = tb
            plsc.store_scatter(es_arr, [tb], _iota16() + 16 * k,
                               mask=cnt > 0)
        mcar = jnp.int32(0)
        for k in range(T // 16):
            ev = es_arr[pl.ds(16 * k, 16)]
            es = jnp.maximum(plsc.cummax(ev), mcar)
            mcar = jnp.max(es)
            tv = _iota16() + 16 * k
            lobk = plsc.load_gather(lob_v, [es])
            tbk = plsc.load_gather(tbase_v, [es])
            offk = plsc.load_gather(basev, [es])
            cntk = plsc.load_gather(cnts, [es])
            rb = lobk + (tv - tbk)
            lo = jnp.maximum(offk, rb * RB)
            hi = jnp.minimum(offk + cntk, rb * RB + RB)
            validv = tv < tcar
            rb = jnp.where(validv, rb, NB - 1)
            lo = jnp.where(validv, lo, 0)
            hi = jnp.where(validv, hi, 0)
            fr = jnp.where(lo == rb * RB, 1, 0)
            meta_v[0, pl.ds(16 * k, 16)] = rb
            meta_v[1, pl.ds(16 * k, 16)] = jnp.where(validv, es, 0)
            meta_v[2, pl.ds(16 * k, 16)] = lo
            meta_v[3, pl.ds(16 * k, 16)] = hi
            meta_v[4, pl.ds(16 * k, 16)] = fr
        pltpu.sync_copy(meta_v, meta_hbm)

    xfetch.wait()
    pltpu.async_copy(xrows, xs_hbm.at[destv], sem).wait()


def _sc_dispatch(xhat, eid):
    mesh = plsc.VectorSubcoreMesh(core_axis_name="c", subcore_axis_name="s")
    hists, ranks = pl.kernel(
        _sc_hist_body,
        out_type=[jax.ShapeDtypeStruct((NW, E), jnp.int32),
                  jax.ShapeDtypeStruct((B,), jnp.int32)],
        mesh=mesh,
        compiler_params=_SC_PARAMS,
        scratch_types=[pltpu.VMEM((NTOK,), jnp.int32),
                       pltpu.VMEM((E,), jnp.int32),
                       pltpu.VMEM((NTOK,), jnp.int32)],
    )(eid)
    outs = pl.kernel(
        _sc_scatter_body,
        out_type=[jax.ShapeDtypeStruct((B, D), _F32),
                  jax.ShapeDtypeStruct((B,), jnp.int32),
                  jax.ShapeDtypeStruct((5, T), jnp.int32)],
        mesh=mesh,
        compiler_params=_SC_PARAMS,
        scratch_types=[pltpu.VMEM((NW, E), jnp.int32),
                       pltpu.VMEM((NTOK,), jnp.int32),
                       pltpu.VMEM((NTOK,), jnp.int32),
                       pltpu.VMEM((NTOK,), jnp.int32),
                       pltpu.VMEM((E,), jnp.int32),
                       pltpu.VMEM((E,), jnp.int32),
                       pltpu.VMEM((E,), jnp.int32),
                       pltpu.VMEM((E,), jnp.int32),
                       pltpu.VMEM((T,), jnp.int32),
                       pltpu.VMEM((5, T), jnp.int32),
                       pltpu.VMEM((NTOK, D), _F32),
                       pltpu.SemaphoreType.DMA],
    )(xhat, eid, hists, ranks)
    return outs[0], outs[1], outs[2]


def _gmm_body(m_ref,
              xs_ref, gamma_ref, beta_ref, W1_ref, b1_ref, W2_ref, b2_ref,
              out_ref):
    t = pl.program_id(0)
    lo = m_ref[2, t]
    hi = m_ref[3, t]

    @pl.when(hi > lo)
    def _compute():
        base = m_ref[0, t] * RB
        es = m_ref[1, t]
        xn = (xs_ref[pl.ds(base, RB), :] * gamma_ref[pl.ds(es, 1), :]
              + beta_ref[pl.ds(es, 1), :])
        h = _gelu(lax.dot_general(xn, W1_ref[0], (((1,), (0,)), ((), ())),
                                  precision=_PREC, preferred_element_type=_F32)
                  + b1_ref[pl.ds(es, 1), :])
        y = lax.dot_general(h, W2_ref[es], (((1,), (0,)), ((), ())),
                            precision=_PREC, preferred_element_type=_F32)
        y = y + b2_ref[pl.ds(es, 1), :]
        row = base + lax.broadcasted_iota(jnp.int32, (RB, 1), 0)
        y = jnp.where((row >= lo) & (row < hi), y, 0.0)
        prev = jnp.where(m_ref[4, t] != 0, 0.0, out_ref[pl.ds(base, RB), :])
        out_ref[pl.ds(base, RB), :] = prev + y


def _gmm(meta, xs, gamma, beta, W1, b1, W2, b2):
    grid_spec = pltpu.PrefetchScalarGridSpec(
        num_scalar_prefetch=1,
        grid=(T,),
        in_specs=[
            pl.BlockSpec((B, D), lambda t, m: (0, 0)),
            pl.BlockSpec((E, D), lambda t, m: (0, 0)),
            pl.BlockSpec((E, D), lambda t, m: (0, 0)),
            pl.BlockSpec((1, D, H), lambda t, m: (m[1, t], 0, 0)),
            pl.BlockSpec((E, H), lambda t, m: (0, 0)),
            pl.BlockSpec((E, H, C), lambda t, m: (0, 0, 0)),
            pl.BlockSpec((E, C), lambda t, m: (0, 0)),
        ],
        out_specs=pl.BlockSpec((B, C), lambda t, m: (0, 0)),
    )
    return pl.pallas_call(
        _gmm_body,
        grid_spec=grid_spec,
        out_shape=jax.ShapeDtypeStruct((B, C), _F32),
    )(meta, xs, gamma, beta, W1, b1, W2, b2)


def _sc_unperm_body(ys_hbm, inv_hbm, out_hbm, ipv, yrows, sem):
    w = lax.axis_index("s") * NC + lax.axis_index("c")
    base = w * NTOK
    pltpu.sync_copy(inv_hbm.at[pl.ds(base, NTOK)], ipv)
    pltpu.async_copy(ys_hbm.at[ipv], yrows, sem).wait()
    pltpu.sync_copy(yrows, out_hbm.at[pl.ds(base, NTOK)])


def _sc_unpermute(ys, invperm):
    mesh = plsc.VectorSubcoreMesh(core_axis_name="c", subcore_axis_name="s")
    return pl.kernel(
        _sc_unperm_body,
        out_type=jax.ShapeDtypeStruct((B, C), _F32),
        mesh=mesh,
        compiler_params=_SC_PARAMS,
        scratch_types=[pltpu.VMEM((NTOK,), jnp.int32),
                       pltpu.VMEM((NTOK, C), _F32),
                       pltpu.SemaphoreType.DMA],
    )(ys, invperm)


def kernel(x, Wr, br, gamma, beta, W1, b1, W2, b2):
    eid, xhat = _router(x, Wr, br)
    xs, invperm, meta = _sc_dispatch(xhat, eid)
    ys = _gmm(meta, xs, gamma, beta, W1, b1, W2, b2)
    return _sc_unpermute(ys, invperm)


# speedup vs baseline: 1.2053x; 1.1608x over previous
import functools

import jax
import jax.numpy as jnp
from jax import lax
from jax.experimental import pallas as pl
from jax.experimental.pallas import tpu as pltpu
from jax.experimental.pallas import tpu_sc as plsc

E = 64
D = 768
H = 128
C = 128
B = 2048
LN_EPS = 1e-5
RB = 128
NB = B // RB
T = NB + E
RBR = 256
NC, NS = 2, 16
NW = NC * NS
NTOK = B // NW

_F32 = jnp.float32
_PREC = lax.Precision.DEFAULT
_SC_PARAMS = pltpu.CompilerParams(needs_layout_passes=False)


def _gelu(v):
    return 0.5 * v * (1.0 + lax.erf(v * 0.7071067811865476))


def _iota16():
    return lax.broadcasted_iota(jnp.int32, (16,), 0)


def _router_body(x_ref, Wr_ref, br_ref, eid_ref, xhat_ref):
    logits = lax.dot_general(x_ref[...], Wr_ref[...], (((1,), (1,)), ((), ())),
                             precision=_PREC, preferred_element_type=_F32)
    logits = logits + br_ref[...]
    eid_ref[...] = jnp.argmax(logits, axis=1).astype(jnp.int32)
    xv = x_ref[...]
    mu = jnp.mean(xv, axis=1, keepdims=True)
    var = jnp.mean((xv - mu) ** 2, axis=1, keepdims=True)
    xhat_ref[...] = (xv - mu) * lax.rsqrt(var + LN_EPS)


def _router(x, Wr, br):
    return pl.pallas_call(
        _router_body,
        grid=(B // RBR,),
        in_specs=[
            pl.BlockSpec((RBR, D), lambda i: (i, 0)),
            pl.BlockSpec((E, D), lambda i: (0, 0)),
            pl.BlockSpec((1, E), lambda i: (0, 0)),
        ],
        out_specs=[pl.BlockSpec((RBR,), lambda i: (i,)),
                   pl.BlockSpec((RBR, D), lambda i: (i, 0))],
        out_shape=[jax.ShapeDtypeStruct((B,), jnp.int32),
                   jax.ShapeDtypeStruct((B, D), _F32)],
    )(x, Wr, br.reshape(1, E))


def _sc_hist_body(eid_hbm, hists_hbm, ranks_hbm, eidv, hist, rank):
    w = lax.axis_index("s") * NC + lax.axis_index("c")
    base = w * NTOK
    pltpu.sync_copy(eid_hbm.at[pl.ds(base, NTOK)], eidv)
    z16 = jnp.zeros((16,), jnp.int32)
    for k in range(E // 16):
        hist[pl.ds(16 * k, 16)] = z16

    lane0 = _iota16() == 0

    def body(i, carry):
        e16 = plsc.load_gather(eidv, [jnp.full((16,), i, jnp.int32)])
        r16 = plsc.load_gather(hist, [e16])
        plsc.store_scatter(rank, [jnp.full((16,), i, jnp.int32)], r16,
                           mask=lane0)
        plsc.store_scatter(hist, [e16], r16 + 1, mask=lane0)
        return carry

    lax.fori_loop(0, NTOK, body, 0)
    pltpu.sync_copy(hist, hists_hbm.at[w])
    pltpu.sync_copy(rank, ranks_hbm.at[pl.ds(base, NTOK)])


def _sc_scatter_body(xhat_hbm, eid_hbm, hists_hbm, ranks_hbm,
                     xs_hbm, invperm_hbm, meta_hbm,
                     allc, eidv, rankv, destv, basev, cnts,
                     lob_v, tbase_v, es_arr, meta_v, xrows, sem):
    w = lax.axis_index("s") * NC + lax.axis_index("c")
    base = w * NTOK
    xfetch = pltpu.async_copy(xhat_hbm.at[pl.ds(base, NTOK)], xrows, sem)
    pltpu.sync_copy(hists_hbm, allc)
    pltpu.sync_copy(eid_hbm.at[pl.ds(base, NTOK)], eidv)
    pltpu.sync_copy(ranks_hbm.at[pl.ds(base, NTOK)], rankv)

    carry = jnp.int32(0)
    for k in range(E // 16):
        z16 = jnp.zeros((16,), jnp.int32)

        def wbody(w2, tm):
            tot, mine = tm
            row = allc[w2, pl.ds(16 * k, 16)]
            tot = tot + row
            mine = mine + jnp.where(w2 < w, row, 0)
            return (tot, mine)

        tot, mine = lax.fori_loop(0, NW, wbody, (z16, z16))
        cs = plsc.cumsum(tot)
        basev[pl.ds(16 * k, 16)] = cs - tot + carry + mine
        cnts[pl.ds(16 * k, 16)] = tot
        carry = carry + jnp.sum(tot)

    for k in range(NTOK // 16):
        ev = eidv[pl.ds(16 * k, 16)]
        bg = plsc.load_gather(basev, [ev])
        destv[pl.ds(16 * k, 16)] = bg + rankv[pl.ds(16 * k, 16)]

    pltpu.sync_copy(destv, invperm_hbm.at[pl.ds(base, NTOK)])

    @pl.when(w == 0)
    def _pair_schedule():
        z16 = jnp.zeros((16,), jnp.int32)
        for k in range(T // 16):
            es_arr[pl.ds(16 * k, 16)] = z16
        tcar = jnp.int32(0)
        for k in range(E // 16):
            off = basev[pl.ds(16 * k, 16)]
            cnt = cnts[pl.ds(16 * k, 16)]
            lob = lax.div(off, RB)
            hib = lax.div(off + cnt - 1, RB)
            nb = jnp.where(cnt > 0, hib - lob + 1, 0)
            cs = plsc.cumsum(nb)
            tb = cs - nb + tcar
            tcar = tcar + jnp.sum(nb)
            lob_v[pl.ds(16 * k, 16)] = lob
            tbase_v[pl.ds(16 * k, 16)] = tb
            plsc.store_scatter(es_arr, [tb], _iota16() + 16 * k,
                               mask=cnt > 0)
        mcar = jnp.int32(0)
        for k in range(T // 16):
            ev = es_arr[pl.ds(16 * k, 16)]
            es = jnp.maximum(plsc.cummax(ev), mcar)
            mcar = jnp.max(es)
            tv = _iota16() + 16 * k
            lobk = plsc.load_gather(lob_v, [es])
            tbk = plsc.load_gather(tbase_v, [es])
            offk = plsc.load_gather(basev, [es])
            cntk = plsc.load_gather(cnts, [es])
            rb = lobk + (tv - tbk)
            lo = jnp.maximum(offk, rb * RB)
            hi = jnp.minimum(offk + cntk, rb * RB + RB)
            validv = tv < tcar
            rb = jnp.where(validv, rb, NB - 1)
            lo = jnp.where(validv, lo, 0)
            hi = jnp.where(validv, hi, 0)
            fr = jnp.where(lo == rb * RB, 1, 0)
            meta_v[0, pl.ds(16 * k, 16)] = rb
            meta_v[1, pl.ds(16 * k, 16)] = jnp.where(validv, es, 0)
            meta_v[2, pl.ds(16 * k, 16)] = lo
            meta_v[3, pl.ds(16 * k, 16)] = hi
            meta_v[4, pl.ds(16 * k, 16)] = fr
        pltpu.sync_copy(meta_v, meta_hbm)

    xfetch.wait()
    pltpu.async_copy(xrows, xs_hbm.at[destv], sem).wait()


def _sc_dispatch(xhat, eid):
    mesh = plsc.VectorSubcoreMesh(core_axis_name="c", subcore_axis_name="s")
    hists, ranks = pl.kernel(
        _sc_hist_body,
        out_type=[jax.ShapeDtypeStruct((NW, E), jnp.int32),
                  jax.ShapeDtypeStruct((B,), jnp.int32)],
        mesh=mesh,
        compiler_params=_SC_PARAMS,
        scratch_types=[pltpu.VMEM((NTOK,), jnp.int32),
                       pltpu.VMEM((E,), jnp.int32),
                       pltpu.VMEM((NTOK,), jnp.int32)],
    )(eid)
    outs = pl.kernel(
        _sc_scatter_body,
        out_type=[jax.ShapeDtypeStruct((B, D), _F32),
                  jax.ShapeDtypeStruct((B,), jnp.int32),
                  jax.ShapeDtypeStruct((5, T), jnp.int32)],
        mesh=mesh,
        compiler_params=_SC_PARAMS,
        scratch_types=[pltpu.VMEM((NW, E), jnp.int32),
                       pltpu.VMEM((NTOK,), jnp.int32),
                       pltpu.VMEM((NTOK,), jnp.int32),
                       pltpu.VMEM((NTOK,), jnp.int32),
                       pltpu.VMEM((E,), jnp.int32),
                       pltpu.VMEM((E,), jnp.int32),
                       pltpu.VMEM((E,), jnp.int32),
                       pltpu.VMEM((E,), jnp.int32),
                       pltpu.VMEM((T,), jnp.int32),
                       pltpu.VMEM((5, T), jnp.int32),
                       pltpu.VMEM((NTOK, D), _F32),
                       pltpu.SemaphoreType.DMA],
    )(xhat, eid, hists, ranks)
    return outs[0], outs[1], outs[2]


def _gmm_body(m_ref,
              xs_ref, gamma_ref, beta_ref, W1_ref, b1_ref, W2_ref, b2_ref,
              out_ref):
    t = pl.program_id(0)
    lo = m_ref[2, t]
    hi = m_ref[3, t]

    @pl.when(hi > lo)
    def _compute():
        base = m_ref[0, t] * RB
        es = m_ref[1, t]
        xn = (xs_ref[pl.ds(base, RB), :] * gamma_ref[pl.ds(es, 1), :]
              + beta_ref[pl.ds(es, 1), :])
        h = _gelu(lax.dot_general(xn, W1_ref[es], (((1,), (0,)), ((), ())),
                                  precision=_PREC, preferred_element_type=_F32)
                  + b1_ref[pl.ds(es, 1), :])
        y = lax.dot_general(h, W2_ref[es], (((1,), (0,)), ((), ())),
                            precision=_PREC, preferred_element_type=_F32)
        y = y + b2_ref[pl.ds(es, 1), :]
        row = base + lax.broadcasted_iota(jnp.int32, (RB, 1), 0)
        y = jnp.where((row >= lo) & (row < hi), y, 0.0)
        prev = jnp.where(m_ref[4, t] != 0, 0.0, out_ref[pl.ds(base, RB), :])
        out_ref[pl.ds(base, RB), :] = prev + y


def _gmm(meta, xs, gamma, beta, W1, b1, W2, b2):
    grid_spec = pltpu.PrefetchScalarGridSpec(
        num_scalar_prefetch=1,
        grid=(T,),
        in_specs=[
            pl.BlockSpec((B, D), lambda t, m: (0, 0)),
            pl.BlockSpec((E, D), lambda t, m: (0, 0)),
            pl.BlockSpec((E, D), lambda t, m: (0, 0)),
            pl.BlockSpec((E, D, H), lambda t, m: (0, 0, 0)),
            pl.BlockSpec((E, H), lambda t, m: (0, 0)),
            pl.BlockSpec((E, H, C), lambda t, m: (0, 0, 0)),
            pl.BlockSpec((E, C), lambda t, m: (0, 0)),
        ],
        out_specs=pl.BlockSpec((B, C), lambda t, m: (0, 0)),
    )
    return pl.pallas_call(
        _gmm_body,
        grid_spec=grid_spec,
        out_shape=jax.ShapeDtypeStruct((B, C), _F32),
    )(meta, xs, gamma, beta, W1, b1, W2, b2)


def _sc_unperm_body(ys_hbm, inv_hbm, out_hbm, ipv, yrows, sem):
    w = lax.axis_index("s") * NC + lax.axis_index("c")
    base = w * NTOK
    pltpu.sync_copy(inv_hbm.at[pl.ds(base, NTOK)], ipv)
    pltpu.async_copy(ys_hbm.at[ipv], yrows, sem).wait()
    pltpu.sync_copy(yrows, out_hbm.at[pl.ds(base, NTOK)])


def _sc_unpermute(ys, invperm):
    mesh = plsc.VectorSubcoreMesh(core_axis_name="c", subcore_axis_name="s")
    return pl.kernel(
        _sc_unperm_body,
        out_type=jax.ShapeDtypeStruct((B, C), _F32),
        mesh=mesh,
        compiler_params=_SC_PARAMS,
        scratch_types=[pltpu.VMEM((NTOK,), jnp.int32),
                       pltpu.VMEM((NTOK, C), _F32),
                       pltpu.SemaphoreType.DMA],
    )(ys, invperm)


def kernel(x, Wr, br, gamma, beta, W1, b1, W2, b2):
    eid, xhat = _router(x, Wr, br)
    xs, invperm, meta = _sc_dispatch(xhat, eid)
    ys = _gmm(meta, xs, gamma, beta, W1, b1, W2, b2)
    return _sc_unpermute(ys, invperm)


# speedup vs baseline: 1.2399x; 1.0288x over previous
import functools

import jax
import jax.numpy as jnp
from jax import lax
from jax.experimental import pallas as pl
from jax.experimental.pallas import tpu as pltpu
from jax.experimental.pallas import tpu_sc as plsc

E = 64
D = 768
H = 128
C = 128
B = 2048
LN_EPS = 1e-5
RB = 128
NB = B // RB
T = NB + E
RBR = 256
NC, NS = 2, 16
NW = NC * NS
NTOK = B // NW

_F32 = jnp.float32
_PREC = lax.Precision.DEFAULT
_SC_PARAMS = pltpu.CompilerParams(needs_layout_passes=False)


def _gelu(v):
    return 0.5 * v * (1.0 + lax.erf(v * 0.7071067811865476))


def _iota16():
    return lax.broadcasted_iota(jnp.int32, (16,), 0)


def _router_body(x_ref, Wr_ref, br_ref, eid_ref, xhat_ref):
    logits = lax.dot_general(x_ref[...], Wr_ref[...], (((1,), (1,)), ((), ())),
                             precision=_PREC, preferred_element_type=_F32)
    logits = logits + br_ref[...]
    eid_ref[...] = jnp.argmax(logits, axis=1).astype(jnp.int32)
    xv = x_ref[...]
    mu = jnp.mean(xv, axis=1, keepdims=True)
    var = jnp.mean((xv - mu) ** 2, axis=1, keepdims=True)
    xhat_ref[...] = (xv - mu) * lax.rsqrt(var + LN_EPS)


def _router(x, Wr, br):
    return pl.pallas_call(
        _router_body,
        grid=(B // RBR,),
        in_specs=[
            pl.BlockSpec((RBR, D), lambda i: (i, 0)),
            pl.BlockSpec((E, D), lambda i: (0, 0)),
            pl.BlockSpec((1, E), lambda i: (0, 0)),
        ],
        out_specs=[pl.BlockSpec((RBR,), lambda i: (i,)),
                   pl.BlockSpec((RBR, D), lambda i: (i, 0))],
        out_shape=[jax.ShapeDtypeStruct((B,), jnp.int32),
                   jax.ShapeDtypeStruct((B, D), _F32)],
    )(x, Wr, br.reshape(1, E))


def _sc_hist_body(eid_hbm, hists_hbm, ranks_hbm, eidv, hist, rank):
    w = lax.axis_index("s") * NC + lax.axis_index("c")
    base = w * NTOK
    pltpu.sync_copy(eid_hbm.at[pl.ds(base, NTOK)], eidv)
    z16 = jnp.zeros((16,), jnp.int32)
    for k in range(E // 16):
        hist[pl.ds(16 * k, 16)] = z16

    lane0 = _iota16() == 0

    def body(i, carry):
        e16 = plsc.load_gather(eidv, [jnp.full((16,), i, jnp.int32)])
        r16 = plsc.load_gather(hist, [e16])
        plsc.store_scatter(rank, [jnp.full((16,), i, jnp.int32)], r16,
                           mask=lane0)
        plsc.store_scatter(hist, [e16], r16 + 1, mask=lane0)
        return carry

    lax.fori_loop(0, NTOK, body, 0)
    pltpu.sync_copy(hist, hists_hbm.at[w])
    pltpu.sync_copy(rank, ranks_hbm.at[pl.ds(base, NTOK)])


def _sc_scatter_body(xhat_hbm, eid_hbm, hists_hbm, ranks_hbm,
                     xs_hbm, invperm_hbm, meta_hbm,
                     allc, eidv, rankv, destv, basev, cnts,
                     lob_v, tbase_v, es_arr, meta_v, xrows, sem):
    w = lax.axis_index("s") * NC + lax.axis_index("c")
    base = w * NTOK
    xfetch = pltpu.async_copy(xhat_hbm.at[pl.ds(base, NTOK)], xrows, sem)
    pltpu.sync_copy(hists_hbm, allc)
    pltpu.sync_copy(eid_hbm.at[pl.ds(base, NTOK)], eidv)
    pltpu.sync_copy(ranks_hbm.at[pl.ds(base, NTOK)], rankv)

    carry = jnp.int32(0)
    for k in range(E // 16):
        z16 = jnp.zeros((16,), jnp.int32)

        def wbody(w2, tm):
            tot, mine = tm
            row = allc[w2, pl.ds(16 * k, 16)]
            tot = tot + row
            mine = mine + jnp.where(w2 < w, row, 0)
            return (tot, mine)

        tot, mine = lax.fori_loop(0, NW, wbody, (z16, z16))
        cs = plsc.cumsum(tot)
        basev[pl.ds(16 * k, 16)] = cs - tot + carry + mine
        cnts[pl.ds(16 * k, 16)] = tot
        carry = carry + jnp.sum(tot)

    for k in range(NTOK // 16):
        ev = eidv[pl.ds(16 * k, 16)]
        bg = plsc.load_gather(basev, [ev])
        destv[pl.ds(16 * k, 16)] = bg + rankv[pl.ds(16 * k, 16)]

    pltpu.sync_copy(destv, invperm_hbm.at[pl.ds(base, NTOK)])

    @pl.when(w == 0)
    def _pair_schedule():
        z16 = jnp.zeros((16,), jnp.int32)
        for k in range(T // 16):
            es_arr[pl.ds(16 * k, 16)] = z16
        tcar = jnp.int32(0)
        for k in range(E // 16):
            off = basev[pl.ds(16 * k, 16)]
            cnt = cnts[pl.ds(16 * k, 16)]
            lob = lax.div(off, RB)
            hib = lax.div(off + cnt - 1, RB)
            nb = jnp.where(cnt > 0, hib - lob + 1, 0)
            cs = plsc.cumsum(nb)
            tb = cs - nb + tcar
            tcar = tcar + jnp.sum(nb)
            lob_v[pl.ds(16 * k, 16)] = lob
            tbase_v[pl.ds(16 * k, 16)] = tb
            plsc.store_scatter(es_arr, [tb], _iota16() + 16 * k,
                               mask=cnt > 0)
        mcar = jnp.int32(0)
        for k in range(T // 16):
            ev = es_arr[pl.ds(16 * k, 16)]
            es = jnp.maximum(plsc.cummax(ev), mcar)
            mcar = jnp.max(es)
            tv = _iota16() + 16 * k
            lobk = plsc.load_gather(lob_v, [es])
            tbk = plsc.load_gather(tbase_v, [es])
            offk = plsc.load_gather(basev, [es])
            cntk = plsc.load_gather(cnts, [es])
            rb = lobk + (tv - tbk)
            lo = jnp.maximum(offk, rb * RB)
            hi = jnp.minimum(offk + cntk, rb * RB + RB)
            validv = tv < tcar
            rb = jnp.where(validv, rb, NB - 1)
            lo = jnp.where(validv, lo, 0)
            hi = jnp.where(validv, hi, 0)
            fr = jnp.where(lo == rb * RB, 1, 0)
            meta_v[0, pl.ds(16 * k, 16)] = rb
            meta_v[1, pl.ds(16 * k, 16)] = jnp.where(validv, es, 0)
            meta_v[2, pl.ds(16 * k, 16)] = lo
            meta_v[3, pl.ds(16 * k, 16)] = hi
            meta_v[4, pl.ds(16 * k, 16)] = fr
        pltpu.sync_copy(meta_v, meta_hbm)

    xfetch.wait()
    pltpu.async_copy(xrows, xs_hbm.at[destv], sem).wait()


def _sc_dispatch(xhat, eid):
    mesh = plsc.VectorSubcoreMesh(core_axis_name="c", subcore_axis_name="s")
    hists, ranks = pl.kernel(
        _sc_hist_body,
        out_type=[jax.ShapeDtypeStruct((NW, E), jnp.int32),
                  jax.ShapeDtypeStruct((B,), jnp.int32)],
        mesh=mesh,
        compiler_params=_SC_PARAMS,
        scratch_types=[pltpu.VMEM((NTOK,), jnp.int32),
                       pltpu.VMEM((E,), jnp.int32),
                       pltpu.VMEM((NTOK,), jnp.int32)],
    )(eid)
    outs = pl.kernel(
        _sc_scatter_body,
        out_type=[jax.ShapeDtypeStruct((B, D), _F32),
                  jax.ShapeDtypeStruct((B,), jnp.int32),
                  jax.ShapeDtypeStruct((5, T), jnp.int32)],
        mesh=mesh,
        compiler_params=_SC_PARAMS,
        scratch_types=[pltpu.VMEM((NW, E), jnp.int32),
                       pltpu.VMEM((NTOK,), jnp.int32),
                       pltpu.VMEM((NTOK,), jnp.int32),
                       pltpu.VMEM((NTOK,), jnp.int32),
                       pltpu.VMEM((E,), jnp.int32),
                       pltpu.VMEM((E,), jnp.int32),
                       pltpu.VMEM((E,), jnp.int32),
                       pltpu.VMEM((E,), jnp.int32),
                       pltpu.VMEM((T,), jnp.int32),
                       pltpu.VMEM((5, T), jnp.int32),
                       pltpu.VMEM((NTOK, D), _F32),
                       pltpu.SemaphoreType.DMA],
    )(xhat, eid, hists, ranks)
    return outs[0], outs[1], outs[2]


NCH = 8
ECH = E // NCH


def _w1_chunk_copy(W1_hbm, W1_vmem, sems, c):
    return pltpu.make_async_copy(W1_hbm.at[pl.ds(c * ECH, ECH)],
                                 W1_vmem.at[pl.ds(c * ECH, ECH)],
                                 sems.at[c])


def _gmm_body(m_ref,
              xs_ref, gamma_ref, beta_ref, W1_hbm, b1_ref, W2_ref, b2_ref,
              out_ref, W1_vmem, sems, wcnt):
    t = pl.program_id(0)
    lo = m_ref[2, t]
    hi = m_ref[3, t]

    @pl.when(t == 0)
    def _issue():
        wcnt[0] = 0
        for c in range(NCH):
            _w1_chunk_copy(W1_hbm, W1_vmem, sems, c).start()

    @pl.when(hi > lo)
    def _compute():
        base = m_ref[0, t] * RB
        es = m_ref[1, t]
        need = es // ECH

        def wbody(w):
            _w1_chunk_copy(W1_hbm, W1_vmem, sems, w).wait()
            return w + 1

        wcnt[0] = lax.while_loop(lambda w: w <= need, wbody, wcnt[0])
        xn = (xs_ref[pl.ds(base, RB), :] * gamma_ref[pl.ds(es, 1), :]
              + beta_ref[pl.ds(es, 1), :])
        h = _gelu(lax.dot_general(xn, W1_vmem[es], (((1,), (0,)), ((), ())),
                                  precision=_PREC, preferred_element_type=_F32)
                  + b1_ref[pl.ds(es, 1), :])
        y = lax.dot_general(h, W2_ref[es], (((1,), (0,)), ((), ())),
                            precision=_PREC, preferred_element_type=_F32)
        y = y + b2_ref[pl.ds(es, 1), :]
        row = base + lax.broadcasted_iota(jnp.int32, (RB, 1), 0)
        y = jnp.where((row >= lo) & (row < hi), y, 0.0)
        prev = jnp.where(m_ref[4, t] != 0, 0.0, out_ref[pl.ds(base, RB), :])
        out_ref[pl.ds(base, RB), :] = prev + y

    @pl.when(t == T - 1)
    def _drain():
        def wbody(w):
            _w1_chunk_copy(W1_hbm, W1_vmem, sems, w).wait()
            return w + 1

        wcnt[0] = lax.while_loop(lambda w: w < NCH, wbody, wcnt[0])


def _gmm(meta, xs, gamma, beta, W1, b1, W2, b2):
    grid_spec = pltpu.PrefetchScalarGridSpec(
        num_scalar_prefetch=1,
        grid=(T,),
        in_specs=[
            pl.BlockSpec((B, D), lambda t, m: (0, 0)),
            pl.BlockSpec((E, D), lambda t, m: (0, 0)),
            pl.BlockSpec((E, D), lambda t, m: (0, 0)),
            pl.BlockSpec(memory_space=pl.ANY),
            pl.BlockSpec((E, H), lambda t, m: (0, 0)),
            pl.BlockSpec((E, H, C), lambda t, m: (0, 0, 0)),
            pl.BlockSpec((E, C), lambda t, m: (0, 0)),
        ],
        out_specs=pl.BlockSpec((B, C), lambda t, m: (0, 0)),
        scratch_shapes=[pltpu.VMEM((E, D, H), _F32),
                        pltpu.SemaphoreType.DMA((NCH,)),
                        pltpu.SMEM((1,), jnp.int32)],
    )
    return pl.pallas_call(
        _gmm_body,
        grid_spec=grid_spec,
        out_shape=jax.ShapeDtypeStruct((B, C), _F32),
    )(meta, xs, gamma, beta, W1, b1, W2, b2)


def _sc_unperm_body(ys_hbm, inv_hbm, out_hbm, ipv, yrows, sem):
    w = lax.axis_index("s") * NC + lax.axis_index("c")
    base = w * NTOK
    pltpu.sync_copy(inv_hbm.at[pl.ds(base, NTOK)], ipv)
    pltpu.async_copy(ys_hbm.at[ipv], yrows, sem).wait()
    pltpu.sync_copy(yrows, out_hbm.at[pl.ds(base, NTOK)])


def _sc_unpermute(ys, invperm):
    mesh = plsc.VectorSubcoreMesh(core_axis_name="c", subcore_axis_name="s")
    return pl.kernel(
        _sc_unperm_body,
        out_type=jax.ShapeDtypeStruct((B, C), _F32),
        mesh=mesh,
        compiler_params=_SC_PARAMS,
        scratch_types=[pltpu.VMEM((NTOK,), jnp.int32),
                       pltpu.VMEM((NTOK, C), _F32),
                       pltpu.SemaphoreType.DMA],
    )(ys, invperm)


def kernel(x, Wr, br, gamma, beta, W1, b1, W2, b2):
    eid, xhat = _router(x, Wr, br)
    xs, invperm, meta = _sc_dispatch(xhat, eid)
    ys = _gmm(meta, xs, gamma, beta, W1, b1, W2, b2)
    return _sc_unpermute(ys, invperm)


# speedup vs baseline: 1.2552x; 1.0123x over previous
import functools

import jax
import jax.numpy as jnp
from jax import lax
from jax.experimental import pallas as pl
from jax.experimental.pallas import tpu as pltpu
from jax.experimental.pallas import tpu_sc as plsc

E = 64
D = 768
H = 128
C = 128
B = 2048
LN_EPS = 1e-5
RB = 128
NB = B // RB
T = NB + E
RBR = 256
NC, NS = 2, 16
NW = NC * NS
NTOK = B // NW

_F32 = jnp.float32
_PREC = lax.Precision.DEFAULT
_SC_PARAMS = pltpu.CompilerParams(needs_layout_passes=False)


def _gelu(v):
    return 0.5 * v * (1.0 + lax.erf(v * 0.7071067811865476))


def _iota16():
    return lax.broadcasted_iota(jnp.int32, (16,), 0)


def _router_body(x_ref, Wr_ref, br_ref, eid_ref, xhat_ref):
    logits = lax.dot_general(x_ref[...], Wr_ref[...], (((1,), (1,)), ((), ())),
                             precision=_PREC, preferred_element_type=_F32)
    logits = logits + br_ref[...]
    eid_ref[...] = jnp.argmax(logits, axis=1).astype(jnp.int32)
    xv = x_ref[...]
    mu = jnp.mean(xv, axis=1, keepdims=True)
    var = jnp.mean((xv - mu) ** 2, axis=1, keepdims=True)
    xhat_ref[...] = (xv - mu) * lax.rsqrt(var + LN_EPS)


def _router(x, Wr, br):
    return pl.pallas_call(
        _router_body,
        grid=(B // RBR,),
        in_specs=[
            pl.BlockSpec((RBR, D), lambda i: (i, 0)),
            pl.BlockSpec((E, D), lambda i: (0, 0)),
            pl.BlockSpec((1, E), lambda i: (0, 0)),
        ],
        out_specs=[pl.BlockSpec((RBR,), lambda i: (i,)),
                   pl.BlockSpec((RBR, D), lambda i: (i, 0))],
        out_shape=[jax.ShapeDtypeStruct((B,), jnp.int32),
                   jax.ShapeDtypeStruct((B, D), _F32)],
    )(x, Wr, br.reshape(1, E))


def _sc_hist_body(eid_hbm, hists_hbm, ranks_hbm, eidv, hist, rank):
    w = lax.axis_index("s") * NC + lax.axis_index("c")
    base = w * NTOK
    pltpu.sync_copy(eid_hbm.at[pl.ds(base, NTOK)], eidv)
    z16 = jnp.zeros((16,), jnp.int32)
    for k in range(E // 16):
        hist[pl.ds(16 * k, 16)] = z16

    lane0 = _iota16() == 0

    def body(i, carry):
        e16 = plsc.load_gather(eidv, [jnp.full((16,), i, jnp.int32)])
        r16 = plsc.load_gather(hist, [e16])
        plsc.store_scatter(rank, [jnp.full((16,), i, jnp.int32)], r16,
                           mask=lane0)
        plsc.store_scatter(hist, [e16], r16 + 1, mask=lane0)
        return carry

    lax.fori_loop(0, NTOK, body, 0)
    pltpu.sync_copy(hist, hists_hbm.at[w])
    pltpu.sync_copy(rank, ranks_hbm.at[pl.ds(base, NTOK)])


def _sc_scatter_body(xhat_hbm, eid_hbm, hists_hbm, ranks_hbm,
                     xs_hbm, invperm_hbm, meta_hbm,
                     allc, eidv, rankv, destv, basev, cnts,
                     lob_v, tbase_v, es_arr, meta_v, xrows, sem):
    w = lax.axis_index("s") * NC + lax.axis_index("c")
    base = w * NTOK
    xfetch = pltpu.async_copy(xhat_hbm.at[pl.ds(base, NTOK)], xrows, sem)
    pltpu.sync_copy(hists_hbm, allc)
    pltpu.sync_copy(eid_hbm.at[pl.ds(base, NTOK)], eidv)
    pltpu.sync_copy(ranks_hbm.at[pl.ds(base, NTOK)], rankv)

    carry = jnp.int32(0)
    for k in range(E // 16):
        z16 = jnp.zeros((16,), jnp.int32)

        def wbody(w2, tm):
            tot, mine = tm
            row = allc[w2, pl.ds(16 * k, 16)]
            tot = tot + row
            mine = mine + jnp.where(w2 < w, row, 0)
            return (tot, mine)

        tot, mine = lax.fori_loop(0, NW, wbody, (z16, z16))
        cs = plsc.cumsum(tot)
        basev[pl.ds(16 * k, 16)] = cs - tot + carry + mine
        cnts[pl.ds(16 * k, 16)] = tot
        carry = carry + jnp.sum(tot)

    for k in range(NTOK // 16):
        ev = eidv[pl.ds(16 * k, 16)]
        bg = plsc.load_gather(basev, [ev])
        destv[pl.ds(16 * k, 16)] = bg + rankv[pl.ds(16 * k, 16)]

    pltpu.sync_copy(destv, invperm_hbm.at[pl.ds(base, NTOK)])

    @pl.when(w == 0)
    def _pair_schedule():
        z16 = jnp.zeros((16,), jnp.int32)
        for k in range(T // 16):
            es_arr[pl.ds(16 * k, 16)] = z16
        tcar = jnp.int32(0)
        for k in range(E // 16):
            off = basev[pl.ds(16 * k, 16)]
            cnt = cnts[pl.ds(16 * k, 16)]
            lob = lax.div(off, RB)
            hib = lax.div(off + cnt - 1, RB)
            nb = jnp.where(cnt > 0, hib - lob + 1, 0)
            cs = plsc.cumsum(nb)
            tb = cs - nb + tcar
            tcar = tcar + jnp.sum(nb)
            lob_v[pl.ds(16 * k, 16)] = lob
            tbase_v[pl.ds(16 * k, 16)] = tb
            plsc.store_scatter(es_arr, [tb], _iota16() + 16 * k,
                               mask=cnt > 0)
        mcar = jnp.int32(0)
        for k in range(T // 16):
            ev = es_arr[pl.ds(16 * k, 16)]
            es = jnp.maximum(plsc.cummax(ev), mcar)
            mcar = jnp.max(es)
            tv = _iota16() + 16 * k
            lobk = plsc.load_gather(lob_v, [es])
            tbk = plsc.load_gather(tbase_v, [es])
            offk = plsc.load_gather(basev, [es])
            cntk = plsc.load_gather(cnts, [es])
            rb = lobk + (tv - tbk)
            lo = jnp.maximum(offk, rb * RB)
            hi = jnp.minimum(offk + cntk, rb * RB + RB)
            validv = tv < tcar
            rb = jnp.where(validv, rb, NB - 1)
            lo = jnp.where(validv, lo, 0)
            hi = jnp.where(validv, hi, 0)
            fr = jnp.where(lo == rb * RB, 1, 0)
            meta_v[0, pl.ds(16 * k, 16)] = rb
            meta_v[1, pl.ds(16 * k, 16)] = jnp.where(validv, es, 0)
            meta_v[2, pl.ds(16 * k, 16)] = lo
            meta_v[3, pl.ds(16 * k, 16)] = hi
            meta_v[4, pl.ds(16 * k, 16)] = fr
        pltpu.sync_copy(meta_v, meta_hbm)

    xfetch.wait()
    pltpu.async_copy(xrows, xs_hbm.at[destv], sem).wait()


def _sc_dispatch(xhat, eid):
    mesh = plsc.VectorSubcoreMesh(core_axis_name="c", subcore_axis_name="s")
    hists, ranks = pl.kernel(
        _sc_hist_body,
        out_type=[jax.ShapeDtypeStruct((NW, E), jnp.int32),
                  jax.ShapeDtypeStruct((B,), jnp.int32)],
        mesh=mesh,
        compiler_params=_SC_PARAMS,
        scratch_types=[pltpu.VMEM((NTOK,), jnp.int32),
                       pltpu.VMEM((E,), jnp.int32),
                       pltpu.VMEM((NTOK,), jnp.int32)],
    )(eid)
    outs = pl.kernel(
        _sc_scatter_body,
        out_type=[jax.ShapeDtypeStruct((B, D), _F32),
                  jax.ShapeDtypeStruct((B,), jnp.int32),
                  jax.ShapeDtypeStruct((5, T), jnp.int32)],
        mesh=mesh,
        compiler_params=_SC_PARAMS,
        scratch_types=[pltpu.VMEM((NW, E), jnp.int32),
                       pltpu.VMEM((NTOK,), jnp.int32),
                       pltpu.VMEM((NTOK,), jnp.int32),
                       pltpu.VMEM((NTOK,), jnp.int32),
                       pltpu.VMEM((E,), jnp.int32),
                       pltpu.VMEM((E,), jnp.int32),
                       pltpu.VMEM((E,), jnp.int32),
                       pltpu.VMEM((E,), jnp.int32),
                       pltpu.VMEM((T,), jnp.int32),
                       pltpu.VMEM((5, T), jnp.int32),
                       pltpu.VMEM((NTOK, D), _F32),
                       pltpu.SemaphoreType.DMA],
    )(xhat, eid, hists, ranks)
    return outs[0], outs[1], outs[2]


NCH = 8
ECH = E // NCH


BCH = B // NCH


def _w1_chunk_copy(W1_hbm, W1_vmem, sems, c):
    return pltpu.make_async_copy(W1_hbm.at[pl.ds(c * ECH, ECH)],
                                 W1_vmem.at[pl.ds(c * ECH, ECH)],
                                 sems.at[c])


def _w2_chunk_copy(W2_hbm, W2_vmem, sems, c):
    return pltpu.make_async_copy(W2_hbm.at[pl.ds(c * ECH, ECH)],
                                 W2_vmem.at[pl.ds(c * ECH, ECH)],
                                 sems.at[c])


def _xs_chunk_copy(xs_hbm, xs_vmem, sems, c):
    return pltpu.make_async_copy(xs_hbm.at[pl.ds(c * BCH, BCH)],
                                 xs_vmem.at[pl.ds(c * BCH, BCH)],
                                 sems.at[c])


def _gmm_body(m_ref,
              xs_hbm, gamma_ref, beta_ref, W1_hbm, b1_ref, W2_hbm, b2_ref,
              out_ref, xs_vmem, W1_vmem, W2_vmem, semx, semw, semv, xcnt, wcnt):
    t = pl.program_id(0)
    lo = m_ref[2, t]
    hi = m_ref[3, t]

    @pl.when(t == 0)
    def _issue():
        xcnt[0] = 0
        wcnt[0] = 0
        for c in range(NCH):
            _xs_chunk_copy(xs_hbm, xs_vmem, semx, c).start()
            _w1_chunk_copy(W1_hbm, W1_vmem, semw, c).start()
            _w2_chunk_copy(W2_hbm, W2_vmem, semv, c).start()

    @pl.when(hi > lo)
    def _compute():
        base = m_ref[0, t] * RB
        es = m_ref[1, t]

        def xbody(w):
            _xs_chunk_copy(xs_hbm, xs_vmem, semx, w).wait()
            return w + 1

        def wbody(w):
            _w1_chunk_copy(W1_hbm, W1_vmem, semw, w).wait()
            _w2_chunk_copy(W2_hbm, W2_vmem, semv, w).wait()
            return w + 1

        xcnt[0] = lax.while_loop(lambda w: w * BCH <= base, xbody, xcnt[0])
        wcnt[0] = lax.while_loop(lambda w: w <= es // ECH, wbody, wcnt[0])
        xn = (xs_vmem[pl.ds(base, RB), :] * gamma_ref[pl.ds(es, 1), :]
              + beta_ref[pl.ds(es, 1), :])
        h = _gelu(lax.dot_general(xn, W1_vmem[es], (((1,), (0,)), ((), ())),
                                  precision=_PREC, preferred_element_type=_F32)
                  + b1_ref[pl.ds(es, 1), :])
        y = lax.dot_general(h, W2_vmem[es], (((1,), (0,)), ((), ())),
                            precision=_PREC, preferred_element_type=_F32)
        y = y + b2_ref[pl.ds(es, 1), :]
        row = base + lax.broadcasted_iota(jnp.int32, (RB, 1), 0)
        y = jnp.where((row >= lo) & (row < hi), y, 0.0)
        prev = jnp.where(m_ref[4, t] != 0, 0.0, out_ref[pl.ds(base, RB), :])
        out_ref[pl.ds(base, RB), :] = prev + y

    @pl.when(t == T - 1)
    def _drain():
        def xbody(w):
            _xs_chunk_copy(xs_hbm, xs_vmem, semx, w).wait()
            return w + 1

        def wbody(w):
            _w1_chunk_copy(W1_hbm, W1_vmem, semw, w).wait()
            _w2_chunk_copy(W2_hbm, W2_vmem, semv, w).wait()
            return w + 1

        xcnt[0] = lax.while_loop(lambda w: w < NCH, xbody, xcnt[0])
        wcnt[0] = lax.while_loop(lambda w: w < NCH, wbody, wcnt[0])


def _gmm(meta, xs, gamma, beta, W1, b1, W2, b2):
    grid_spec = pltpu.PrefetchScalarGridSpec(
        num_scalar_prefetch=1,
        grid=(T,),
        in_specs=[
            pl.BlockSpec(memory_space=pl.ANY),
            pl.BlockSpec((E, D), lambda t, m: (0, 0)),
            pl.BlockSpec((E, D), lambda t, m: (0, 0)),
            pl.BlockSpec(memory_space=pl.ANY),
            pl.BlockSpec((E, H), lambda t, m: (0, 0)),
            pl.BlockSpec(memory_space=pl.ANY),
            pl.BlockSpec((E, C), lambda t, m: (0, 0)),
        ],
        out_specs=pl.BlockSpec((B, C), lambda t, m: (0, 0)),
        scratch_shapes=[pltpu.VMEM((B, D), _F32),
                        pltpu.VMEM((E, D, H), _F32),
                        pltpu.VMEM((E, H, C), _F32),
                        pltpu.SemaphoreType.DMA((NCH,)),
                        pltpu.SemaphoreType.DMA((NCH,)),
                        pltpu.SemaphoreType.DMA((NCH,)),
                        pltpu.SMEM((1,), jnp.int32),
                        pltpu.SMEM((1,), jnp.int32)],
    )
    return pl.pallas_call(
        _gmm_body,
        grid_spec=grid_spec,
        out_shape=jax.ShapeDtypeStruct((B, C), _F32),
    )(meta, xs, gamma, beta, W1, b1, W2, b2)


def _sc_unperm_body(ys_hbm, inv_hbm, out_hbm, ipv, yrows, sem):
    w = lax.axis_index("s") * NC + lax.axis_index("c")
    base = w * NTOK
    pltpu.sync_copy(inv_hbm.at[pl.ds(base, NTOK)], ipv)
    pltpu.async_copy(ys_hbm.at[ipv], yrows, sem).wait()
    pltpu.sync_copy(yrows, out_hbm.at[pl.ds(base, NTOK)])


def _sc_unpermute(ys, invperm):
    mesh = plsc.VectorSubcoreMesh(core_axis_name="c", subcore_axis_name="s")
    return pl.kernel(
        _sc_unperm_body,
        out_type=jax.ShapeDtypeStruct((B, C), _F32),
        mesh=mesh,
        compiler_params=_SC_PARAMS,
        scratch_types=[pltpu.VMEM((NTOK,), jnp.int32),
                       pltpu.VMEM((NTOK, C), _F32),
                       pltpu.SemaphoreType.DMA],
    )(ys, invperm)


def kernel(x, Wr, br, gamma, beta, W1, b1, W2, b2):
    eid, xhat = _router(x, Wr, br)
    xs, invperm, meta = _sc_dispatch(xhat, eid)
    ys = _gmm(meta, xs, gamma, beta, W1, b1, W2, b2)
    return _sc_unpermute(ys, invperm)
